# baseline (speedup 1.0000x reference)
import functools

import jax
import jax.numpy as jnp
from jax import lax
from jax.experimental import pallas as pl
from jax.experimental.pallas import tpu as pltpu
from jax.experimental.pallas import tpu_sc as plsc

F32 = jnp.float32
HIGH = lax.Precision.HIGHEST

H = 16
HH = H * H
PK = 8
NC = 2
NS = 16
NW = NC * NS
CH = 128
NG = 64


def _dot(a, b):
    return jnp.dot(a, b, preferred_element_type=F32)


def _kron8(w):
    return jnp.kron(jnp.eye(PK, dtype=w.dtype), w)


def _tile8(b):
    return jnp.tile(b.reshape(1, -1), (1, PK)).reshape(1, -1)


def _proj_body(nf, w, b, out):
    x = nf[...]
    np_ = out.shape[0]
    parts = [_dot(x[np_ * j:np_ * (j + 1), :], w[...]) for j in range(PK)]
    out[...] = jnp.concatenate(parts, axis=1) + b[...]


def _dotd(a, b):
    return jnp.dot(a, b, preferred_element_type=F32)


def _repack_body(e0, e1, e2, e3, e4, e5, e6, e7, out):
    out[...] = jnp.concatenate(
        [e0[...], e1[...], e2[...], e3[...], e4[...], e5[...], e6[...],
         e7[...]], axis=1)


def _msgs_body(ef, hs, w1b, b1t, w2bp, b2tp, sbp, out):
    eh = jnp.maximum(_dotd(ef[...], w1b[...]) + b1t[...], 0.0)
    a = _dotd(eh, w2bp[...]) + b2tp[...]
    h_ = hs[...]
    hrep = jnp.concatenate([h_] * H, axis=1)
    out[...] = _dotd(a * hrep, sbp[...])


def _gru_core(ma, mb, h, wz, uz, bz, wr, ur, br, wn, un, bn):
    m = ma + mb
    z = jax.nn.sigmoid(_dot(m, wz) + _dot(h, uz) + bz)
    r = jax.nn.sigmoid(_dot(m, wr) + _dot(h, ur) + br)
    n = jnp.tanh(_dot(m, wn) + r * _dot(h, un) + bn)
    return (1.0 - z) * n + z * h


def _gru_body(ma, mb, h, wz, uz, bz, wr, ur, br, wn, un, bn, out):
    np_ = h.shape[0]
    out[...] = _gru_core(ma[0, :np_], mb[0, :np_], h[...],
                         wz[...], uz[...], bz[...],
                         wr[...], ur[...], br[...], wn[...], un[...], bn[...])


def _gru_out_body(ma, mb, h, wz, uz, bz, wr, ur, br, wn, un, bn, wob, oout):
    np_ = h.shape[0]
    hn = _gru_core(ma[0, :np_], mb[0, :np_], h[...],
                   wz[...], uz[...], bz[...],
                   wr[...], ur[...], br[...], wn[...], un[...], bn[...])
    oout[...] = _dot(hn, wob[...])


def _readout_body(o3, b3, bo, g, acc):
    i = pl.program_id(0)
    nb = pl.num_programs(0)
    tn = o3.shape[2]
    rows = lax.broadcasted_iota(jnp.int32, (NG, tn), 0)
    oh = (b3[0] == rows).astype(F32)
    contrib = jnp.sum(oh * o3[0], axis=1, keepdims=True)
    cnt = jnp.sum(oh, axis=1, keepdims=True)

    @pl.when(i == 0)
    def _():
        acc[...] = jnp.zeros_like(acc)

    acc[...] += contrib + cnt * bo[0, 0]

    @pl.when(i == nb - 1)
    def _():
        g[...] = acc[...]


def _make_sc_gather(n_nodes, e_pad, epw, nchunk):
    mesh = plsc.VectorSubcoreMesh(core_axis_name="c", subcore_axis_name="s",
                                  num_cores=NC, num_subcores=NS)

    @functools.partial(
        pl.kernel,
        mesh=mesh,
        out_type=jax.ShapeDtypeStruct((e_pad, H), F32),
        scratch_types=[
            pltpu.VMEM((nchunk, CH), jnp.int32),
            pltpu.VMEM((epw, H), F32),
            pltpu.SemaphoreType.DMA,
            pltpu.SemaphoreType.DMA,
        ],
        compiler_params=pltpu.CompilerParams(use_tc_tiling_on_sc=False),
    )
    def sc_gather(h_hbm, idx_hbm, out_hbm, idx_v, rows_v, sem0, sem1):
        c = lax.axis_index("c")
        s = lax.axis_index("s")
        wid = s * NC + c
        pltpu.sync_copy(idx_hbm.at[wid], idx_v)

        def fire(j, sem):
            pltpu.async_copy(
                h_hbm.at[idx_v.at[j]],
                rows_v.at[pl.ds(j * CH, CH), :],
                sem,
            )

        def drain(sem):
            pltpu.make_async_copy(
                h_hbm.at[idx_v.at[0]],
                rows_v.at[pl.ds(0, CH), :],
                sem,
            ).wait()

        fire(0, sem0)
        fire(1, sem1)

        def body(jj, carry):
            j = jj * 2
            drain(sem0)
            fire(j + 2, sem0)
            drain(sem1)
            fire(j + 3, sem1)
            return carry

        lax.fori_loop(0, nchunk // 2 - 1, body, 0)
        drain(sem0)
        drain(sem1)
        pltpu.sync_copy(rows_v, out_hbm.at[pl.ds(wid * epw, epw), :])

    return sc_gather


def _make_sc_scatter(m_pad, e_pad, epw, nchunk):
    mesh = plsc.VectorSubcoreMesh(core_axis_name="c", subcore_axis_name="s",
                                  num_cores=NC, num_subcores=NS)
    rpt = m_pad // NS

    @functools.partial(
        pl.kernel,
        mesh=mesh,
        out_type=jax.ShapeDtypeStruct((NC, m_pad, H), F32),
        scratch_types=[
            pltpu.VMEM((nchunk, CH), jnp.int32),
            pltpu.VMEM((epw, H), F32),
            pltpu.VMEM_SHARED((m_pad, H), F32),
        ],
        compiler_params=pltpu.CompilerParams(use_tc_tiling_on_sc=False),
    )
    def sc_scatter(msgs_hbm, idx_hbm, zeros_hbm, out_hbm, idx_v, msg_v, macc):
        c = lax.axis_index("c")
        s = lax.axis_index("s")
        wid = s * NC + c

        @pl.when(s == 0)
        def _():
            pltpu.sync_copy(zeros_hbm, macc)

        pltpu.sync_copy(idx_hbm.at[wid], idx_v)
        pltpu.sync_copy(msgs_hbm.at[pl.ds(wid * epw, epw), :], msg_v)
        plsc.subcore_barrier()

        def body(j, carry):
            pltpu.sync_copy(
                msg_v.at[pl.ds(j * CH, CH), :],
                macc.at[idx_v.at[j]],
                add=True,
            )
            return carry

        lax.fori_loop(0, nchunk, body, 0)
        plsc.subcore_barrier()

        pltpu.sync_copy(macc.at[pl.ds(s * rpt, rpt), :],
                        msg_v.at[pl.ds(0, rpt), :])
        pltpu.sync_copy(msg_v.at[pl.ds(0, rpt), :],
                        out_hbm.at[c, pl.ds(s * rpt, rpt), :])

    return sc_scatter


def kernel(node_features, edge_features, Esrc, Etgt, batch, W_in, b_in,
           ee_W1, ee_b1, ee_W2, ee_b2, Wz, Uz, bz, Wr, Ur, br, Wn, Un, bn,
           W_out, b_out):
    n, f = node_features.shape
    e, de = edge_features.shape
    t_rounds = 3

    e_pad = ((e + NW * CH - 1) // (NW * CH)) * (NW * CH)
    epw = e_pad // NW
    nchunk = epw // CH
    m_pad = ((n + 1 + NS * PK - 1) // (NS * PK)) * (NS * PK)
    np_ = n // PK
    mp_ = m_pad // PK
    ep_ = e_pad // PK
    tn = 2000
    nb = n // tn
    tnp = np_ // nb
    tep = 512
    neb = ep_ // tep

    del tnp
    esrc_r = PK * (Esrc % np_) + Esrc // np_
    etgt_r = PK * (Etgt % np_) + Etgt // np_
    ee8 = e // PK
    ss = jnp.arange(e)
    sigma = (ss // PK) + ee8 * (ss % PK)
    esrc_p = jnp.pad(esrc_r[sigma], (0, e_pad - e)).reshape(NW, nchunk, CH)
    etgt_p = jnp.pad(etgt_r[sigma], (0, e_pad - e), constant_values=n).reshape(
        NW, nchunk, CH)
    zeros_m = jnp.zeros((m_pad, H), F32)
    pp = jnp.arange(n)
    batch3 = batch[(pp // PK) + np_ * (pp % PK)].reshape(nb, 1, tn)

    w1b = _kron8(ee_W1)
    w2b = _kron8(ee_W2)
    ll = jnp.arange(PK * HH)
    tt, qq = ll // (PK * H), ll % (PK * H)
    c_orig = HH * (qq // H) + H * tt + qq % H
    w2bp = w2b[:, c_orig]
    colidx = H * (qq // H) + tt
    sbp = (colidx[:, None] == jnp.arange(PK * H)[None, :]).astype(F32)
    uzb, urb, unb = _kron8(Uz), _kron8(Ur), _kron8(Un)
    wzb, wrb, wnb = _kron8(Wz), _kron8(Wr), _kron8(Wn)
    wob = _kron8(W_out)
    b_int = _tile8(b_in)
    b1t = _tile8(ee_b1)
    b2tp = _tile8(ee_b2)[:, c_orig]
    bzt, brt, bnt = _tile8(bz), _tile8(br), _tile8(bn)
    b_out2 = b_out.reshape(1, 1)

    full = lambda shape: pl.BlockSpec(shape, lambda i: tuple(0 for _ in shape))

    h = pl.pallas_call(
        _proj_body,
        grid=(1,),
        in_specs=[
            full((n, f)),
            full((f, H)),
            full((1, PK * H)),
        ],
        out_specs=full((np_, PK * H)),
        out_shape=jax.ShapeDtypeStruct((np_, PK * H), F32),
    )(node_features, W_in, b_int)

    tq = 200
    nqb = ee8 // tq
    ef_p = pl.pallas_call(
        _repack_body,
        grid=(nqb,),
        in_specs=[
            pl.BlockSpec((tq, de), (lambda i, j=j: (i + nqb * j, 0)))
            for j in range(PK)
        ],
        out_specs=pl.BlockSpec((tq, PK * de), lambda i: (i, 0)),
        out_shape=jax.ShapeDtypeStruct((ep_, PK * de), F32),
    )(*([edge_features] * PK))

    sc_gather = _make_sc_gather(n, e_pad, epw, nchunk)
    sc_scatter = _make_sc_scatter(m_pad, e_pad, epw, nchunk)

    msgs_call = pl.pallas_call(
        _msgs_body,
        grid=(neb,),
        in_specs=[
            pl.BlockSpec((tep, PK * de), lambda i: (i, 0)),
            pl.BlockSpec((tep, PK * H), lambda i: (i, 0)),
            full((PK * de, PK * H)),
            full((1, PK * H)),
            full((PK * H, PK * HH)),
            full((1, PK * HH)),
            full((PK * HH, PK * H)),
        ],
        out_specs=pl.BlockSpec((tep, PK * H), lambda i: (i, 0)),
        out_shape=jax.ShapeDtypeStruct((ep_, PK * H), F32),
    )

    gru_in_specs = [
        pl.BlockSpec((1, mp_, PK * H), lambda i: (0, 0, 0)),
        pl.BlockSpec((1, mp_, PK * H), lambda i: (1, 0, 0)),
        full((np_, PK * H)),
        full((PK * H, PK * H)), full((PK * H, PK * H)), full((1, PK * H)),
        full((PK * H, PK * H)), full((PK * H, PK * H)), full((1, PK * H)),
        full((PK * H, PK * H)), full((PK * H, PK * H)), full((1, PK * H)),
    ]
    gru_call = pl.pallas_call(
        _gru_body,
        grid=(1,),
        in_specs=gru_in_specs,
        out_specs=full((np_, PK * H)),
        out_shape=jax.ShapeDtypeStruct((np_, PK * H), F32),
    )
    gru_out_call = pl.pallas_call(
        _gru_out_body,
        grid=(1,),
        in_specs=gru_in_specs + [full((PK * H, PK))],
        out_specs=full((np_, PK)),
        out_shape=jax.ShapeDtypeStruct((np_, PK), F32),
    )
    readout_call = pl.pallas_call(
        _readout_body,
        grid=(nb,),
        in_specs=[
            pl.BlockSpec((1, 1, tn), lambda i: (i, 0, 0)),
            pl.BlockSpec((1, 1, tn), lambda i: (i, 0, 0)),
            full((1, 1)),
        ],
        out_specs=pl.BlockSpec((NG, 1), lambda i: (0, 0)),
        out_shape=jax.ShapeDtypeStruct((NG, 1), F32),
        scratch_shapes=[pltpu.VMEM((NG, 1), F32)],
    )

    o_p = None
    for t in range(t_rounds):
        hs = sc_gather(h.reshape(n, H), esrc_p)
        msgs = msgs_call(ef_p, hs.reshape(ep_, PK * H),
                         w1b, b1t, w2bp, b2tp, sbp)
        m2 = sc_scatter(msgs.reshape(e_pad, H), etgt_p, zeros_m)
        m2p = m2.reshape(NC, mp_, PK * H)
        gru_args = (m2p, m2p, h, wzb, uzb, bzt, wrb, urb, brt, wnb, unb, bnt)
        if t < t_rounds - 1:
            h = gru_call(*gru_args)
        else:
            o_p = gru_out_call(*gru_args, wob)
    o3 = o_p.reshape(nb, 1, tn)
    return readout_call(o3, batch3, b_out2)


# speedup vs baseline: 1.1280x; 1.1280x over previous
import functools

import jax
import jax.numpy as jnp
from jax import lax
from jax.experimental import pallas as pl
from jax.experimental.pallas import tpu as pltpu
from jax.experimental.pallas import tpu_sc as plsc

F32 = jnp.float32
HIGH = lax.Precision.HIGHEST

H = 16
HH = H * H
PK = 8
NC = 2
NS = 16
NW = NC * NS
CH = 128
NG = 64


def _dot(a, b):
    return jnp.dot(a, b, preferred_element_type=F32)


def _kron8(w):
    return jnp.kron(jnp.eye(PK, dtype=w.dtype), w)


def _tile8(b):
    return jnp.tile(b.reshape(1, -1), (1, PK)).reshape(1, -1)


def _proj_body(nf, w, b, out):
    x = nf[...]
    np_ = out.shape[0]
    parts = [_dot(x[np_ * j:np_ * (j + 1), :], w[...]) for j in range(PK)]
    out[...] = jnp.concatenate(parts, axis=1) + b[...]


def _dotd(a, b):
    return jnp.dot(a, b, preferred_element_type=F32)


def _msgs_body(ef, hs, w1b, b1t, w2bp, b2tp, sbp, out):
    eh = jnp.maximum(_dotd(ef[...], w1b[...]) + b1t[...], 0.0)
    a = _dotd(eh, w2bp[...]) + b2tp[...]
    h_ = hs[...]
    hrep = jnp.concatenate([h_] * H, axis=1)
    out[...] = _dotd(a * hrep, sbp[...])


def _gru_core(ma, mb, h, wz, uz, bz, wr, ur, br, wn, un, bn):
    m = ma + mb
    z = jax.nn.sigmoid(_dot(m, wz) + _dot(h, uz) + bz)
    r = jax.nn.sigmoid(_dot(m, wr) + _dot(h, ur) + br)
    n = jnp.tanh(_dot(m, wn) + r * _dot(h, un) + bn)
    return (1.0 - z) * n + z * h


def _gru_body(ma, mb, h, wz, uz, bz, wr, ur, br, wn, un, bn, out):
    np_ = h.shape[0]
    out[...] = _gru_core(ma[0, :np_], mb[0, :np_], h[...],
                         wz[...], uz[...], bz[...],
                         wr[...], ur[...], br[...], wn[...], un[...], bn[...])


def _gru_out_body(ma, mb, h, wz, uz, bz, wr, ur, br, wn, un, bn, wob, oout):
    np_ = h.shape[0]
    hn = _gru_core(ma[0, :np_], mb[0, :np_], h[...],
                   wz[...], uz[...], bz[...],
                   wr[...], ur[...], br[...], wn[...], un[...], bn[...])
    oout[...] = _dot(hn, wob[...])


def _readout_body(o3, b3, bo, g, acc):
    i = pl.program_id(0)
    nb = pl.num_programs(0)
    tn = o3.shape[2]
    rows = lax.broadcasted_iota(jnp.int32, (NG, tn), 0)
    oh = (b3[0] == rows).astype(F32)
    contrib = jnp.sum(oh * o3[0], axis=1, keepdims=True)
    cnt = jnp.sum(oh, axis=1, keepdims=True)

    @pl.when(i == 0)
    def _():
        acc[...] = jnp.zeros_like(acc)

    acc[...] += contrib + cnt * bo[0, 0]

    @pl.when(i == nb - 1)
    def _():
        g[...] = acc[...]


def _make_sc_gather(n_nodes, e_pad, epw, nchunk):
    mesh = plsc.VectorSubcoreMesh(core_axis_name="c", subcore_axis_name="s",
                                  num_cores=NC, num_subcores=NS)

    @functools.partial(
        pl.kernel,
        mesh=mesh,
        out_type=jax.ShapeDtypeStruct((e_pad, H), F32),
        scratch_types=[
            pltpu.VMEM((nchunk, CH), jnp.int32),
            pltpu.VMEM((epw, H), F32),
            pltpu.SemaphoreType.DMA,
            pltpu.SemaphoreType.DMA,
        ],
        compiler_params=pltpu.CompilerParams(use_tc_tiling_on_sc=False),
    )
    def sc_gather(h_hbm, idx_hbm, out_hbm, idx_v, rows_v, sem0, sem1):
        c = lax.axis_index("c")
        s = lax.axis_index("s")
        wid = s * NC + c
        pltpu.sync_copy(idx_hbm.at[wid], idx_v)

        def fire(j, sem):
            pltpu.async_copy(
                h_hbm.at[idx_v.at[j]],
                rows_v.at[pl.ds(j * CH, CH), :],
                sem,
            )

        def drain(sem):
            pltpu.make_async_copy(
                h_hbm.at[idx_v.at[0]],
                rows_v.at[pl.ds(0, CH), :],
                sem,
            ).wait()

        fire(0, sem0)
        fire(1, sem1)

        def body(jj, carry):
            j = jj * 2
            drain(sem0)
            fire(j + 2, sem0)
            drain(sem1)
            fire(j + 3, sem1)
            return carry

        lax.fori_loop(0, nchunk // 2 - 1, body, 0)
        drain(sem0)
        drain(sem1)
        pltpu.sync_copy(rows_v, out_hbm.at[pl.ds(wid * epw, epw), :])

    return sc_gather


def _make_sc_scatter(m_pad, e_pad, epw, nchunk):
    mesh = plsc.VectorSubcoreMesh(core_axis_name="c", subcore_axis_name="s",
                                  num_cores=NC, num_subcores=NS)
    rpt = m_pad // NS

    @functools.partial(
        pl.kernel,
        mesh=mesh,
        out_type=jax.ShapeDtypeStruct((NC, m_pad, H), F32),
        scratch_types=[
            pltpu.VMEM((nchunk, CH), jnp.int32),
            pltpu.VMEM((epw, H), F32),
            pltpu.VMEM_SHARED((m_pad, H), F32),
        ],
        compiler_params=pltpu.CompilerParams(use_tc_tiling_on_sc=False),
    )
    def sc_scatter(msgs_hbm, idx_hbm, zeros_hbm, out_hbm, idx_v, msg_v, macc):
        c = lax.axis_index("c")
        s = lax.axis_index("s")
        wid = s * NC + c

        @pl.when(s == 0)
        def _():
            pltpu.sync_copy(zeros_hbm, macc)

        pltpu.sync_copy(idx_hbm.at[wid], idx_v)
        pltpu.sync_copy(msgs_hbm.at[pl.ds(wid * epw, epw), :], msg_v)
        plsc.subcore_barrier()

        def body(j, carry):
            pltpu.sync_copy(
                msg_v.at[pl.ds(j * CH, CH), :],
                macc.at[idx_v.at[j]],
                add=True,
            )
            return carry

        lax.fori_loop(0, nchunk, body, 0)
        plsc.subcore_barrier()

        pltpu.sync_copy(macc.at[pl.ds(s * rpt, rpt), :],
                        msg_v.at[pl.ds(0, rpt), :])
        pltpu.sync_copy(msg_v.at[pl.ds(0, rpt), :],
                        out_hbm.at[c, pl.ds(s * rpt, rpt), :])

    return sc_scatter


def kernel(node_features, edge_features, Esrc, Etgt, batch, W_in, b_in,
           ee_W1, ee_b1, ee_W2, ee_b2, Wz, Uz, bz, Wr, Ur, br, Wn, Un, bn,
           W_out, b_out):
    n, f = node_features.shape
    e, de = edge_features.shape
    t_rounds = 3

    e_pad = ((e + NW * CH - 1) // (NW * CH)) * (NW * CH)
    epw = e_pad // NW
    nchunk = epw // CH
    m_pad = ((n + 1 + NS * PK - 1) // (NS * PK)) * (NS * PK)
    np_ = n // PK
    mp_ = m_pad // PK
    ep_ = e_pad // PK
    tn = 2000
    nb = n // tn
    tnp = np_ // nb
    tep = 512
    neb = ep_ // tep

    del tnp
    esrc_r = PK * (Esrc % np_) + Esrc // np_
    etgt_r = PK * (Etgt % np_) + Etgt // np_
    ef_p = jnp.pad(edge_features.reshape(e // PK, PK * de),
                   ((0, ep_ - e // PK), (0, 0)))
    esrc_p = jnp.pad(esrc_r, (0, e_pad - e)).reshape(NW, nchunk, CH)
    etgt_p = jnp.pad(etgt_r, (0, e_pad - e), constant_values=n).reshape(
        NW, nchunk, CH)
    zeros_m = jnp.zeros((m_pad, H), F32)
    pp = jnp.arange(n)
    batch3 = batch[(pp // PK) + np_ * (pp % PK)].reshape(nb, 1, tn)

    w1b = _kron8(ee_W1)
    w2b = _kron8(ee_W2)
    ll = jnp.arange(PK * HH)
    tt, qq = ll // (PK * H), ll % (PK * H)
    c_orig = HH * (qq // H) + H * tt + qq % H
    w2bp = w2b[:, c_orig]
    colidx = H * (qq // H) + tt
    sbp = (colidx[:, None] == jnp.arange(PK * H)[None, :]).astype(F32)
    uzb, urb, unb = _kron8(Uz), _kron8(Ur), _kron8(Un)
    wzb, wrb, wnb = _kron8(Wz), _kron8(Wr), _kron8(Wn)
    wob = _kron8(W_out)
    b_int = _tile8(b_in)
    b1t = _tile8(ee_b1)
    b2tp = _tile8(ee_b2)[:, c_orig]
    bzt, brt, bnt = _tile8(bz), _tile8(br), _tile8(bn)
    b_out2 = b_out.reshape(1, 1)

    full = lambda shape: pl.BlockSpec(shape, lambda i: tuple(0 for _ in shape))

    h = pl.pallas_call(
        _proj_body,
        grid=(1,),
        in_specs=[
            full((n, f)),
            full((f, H)),
            full((1, PK * H)),
        ],
        out_specs=full((np_, PK * H)),
        out_shape=jax.ShapeDtypeStruct((np_, PK * H), F32),
    )(node_features, W_in, b_int)

    sc_gather = _make_sc_gather(n, e_pad, epw, nchunk)
    sc_scatter = _make_sc_scatter(m_pad, e_pad, epw, nchunk)

    msgs_call = pl.pallas_call(
        _msgs_body,
        grid=(neb,),
        in_specs=[
            pl.BlockSpec((tep, PK * de), lambda i: (i, 0)),
            pl.BlockSpec((tep, PK * H), lambda i: (i, 0)),
            full((PK * de, PK * H)),
            full((1, PK * H)),
            full((PK * H, PK * HH)),
            full((1, PK * HH)),
            full((PK * HH, PK * H)),
        ],
        out_specs=pl.BlockSpec((tep, PK * H), lambda i: (i, 0)),
        out_shape=jax.ShapeDtypeStruct((ep_, PK * H), F32),
    )

    gru_in_specs = [
        pl.BlockSpec((1, mp_, PK * H), lambda i: (0, 0, 0)),
        pl.BlockSpec((1, mp_, PK * H), lambda i: (1, 0, 0)),
        full((np_, PK * H)),
        full((PK * H, PK * H)), full((PK * H, PK * H)), full((1, PK * H)),
        full((PK * H, PK * H)), full((PK * H, PK * H)), full((1, PK * H)),
        full((PK * H, PK * H)), full((PK * H, PK * H)), full((1, PK * H)),
    ]
    gru_call = pl.pallas_call(
        _gru_body,
        grid=(1,),
        in_specs=gru_in_specs,
        out_specs=full((np_, PK * H)),
        out_shape=jax.ShapeDtypeStruct((np_, PK * H), F32),
    )
    gru_out_call = pl.pallas_call(
        _gru_out_body,
        grid=(1,),
        in_specs=gru_in_specs + [full((PK * H, PK))],
        out_specs=full((np_, PK)),
        out_shape=jax.ShapeDtypeStruct((np_, PK), F32),
    )
    readout_call = pl.pallas_call(
        _readout_body,
        grid=(nb,),
        in_specs=[
            pl.BlockSpec((1, 1, tn), lambda i: (i, 0, 0)),
            pl.BlockSpec((1, 1, tn), lambda i: (i, 0, 0)),
            full((1, 1)),
        ],
        out_specs=pl.BlockSpec((NG, 1), lambda i: (0, 0)),
        out_shape=jax.ShapeDtypeStruct((NG, 1), F32),
        scratch_shapes=[pltpu.VMEM((NG, 1), F32)],
    )

    o_p = None
    for t in range(t_rounds):
        hs = sc_gather(h.reshape(n, H), esrc_p)
        msgs = msgs_call(ef_p, hs.reshape(ep_, PK * H),
                         w1b, b1t, w2bp, b2tp, sbp)
        m2 = sc_scatter(msgs.reshape(e_pad, H), etgt_p, zeros_m)
        m2p = m2.reshape(NC, mp_, PK * H)
        gru_args = (m2p, m2p, h, wzb, uzb, bzt, wrb, urb, brt, wnb, unb, bnt)
        if t < t_rounds - 1:
            h = gru_call(*gru_args)
        else:
            o_p = gru_out_call(*gru_args, wob)
    o3 = o_p.reshape(nb, 1, tn)
    return readout_call(o3, batch3, b_out2)


# speedup vs baseline: 1.3413x; 1.1891x over previous
import functools

import jax
import jax.numpy as jnp
from jax import lax
from jax.experimental import pallas as pl
from jax.experimental.pallas import tpu as pltpu
from jax.experimental.pallas import tpu_sc as plsc

F32 = jnp.float32
HIGH = lax.Precision.HIGHEST

H = 16
HH = H * H
PK = 8
NC = 2
NS = 16
NW = NC * NS
CH = 128
NG = 64


def _dot(a, b):
    return jnp.dot(a, b, preferred_element_type=F32)


def _kron8(w):
    return jnp.kron(jnp.eye(PK, dtype=w.dtype), w)


def _tile8(b):
    return jnp.tile(b.reshape(1, -1), (1, PK)).reshape(1, -1)


def _proj_body(nf, w, b, out):
    x = nf[...]
    np_ = out.shape[0]
    parts = [_dot(x[np_ * j:np_ * (j + 1), :], w[...]) for j in range(PK)]
    out[...] = jnp.concatenate(parts, axis=1) + b[...]


def _dotd(a, b):
    return jnp.dot(a, b, preferred_element_type=F32)


def _msgs_body(ef, hs, w1b, b1t, w2bp, b2tp, sbp, out):
    eh = jnp.maximum(_dotd(ef[...], w1b[...]) + b1t[...], 0.0)
    a = _dotd(eh, w2bp[...]) + b2tp[...]
    h_ = hs[...]
    hrep = jnp.concatenate([h_] * H, axis=1)
    out[...] = _dotd(a * hrep, sbp[...])


def _gru_core(ma, mb, h, wz, uz, bz, wr, ur, br, wn, un, bn):
    m = ma + mb
    z = jax.nn.sigmoid(_dot(m, wz) + _dot(h, uz) + bz)
    r = jax.nn.sigmoid(_dot(m, wr) + _dot(h, ur) + br)
    n = jnp.tanh(_dot(m, wn) + r * _dot(h, un) + bn)
    return (1.0 - z) * n + z * h


def _gru_body(ma, mb, h, wz, uz, bz, wr, ur, br, wn, un, bn, out):
    np_ = h.shape[0]
    out[...] = _gru_core(ma[0, :np_], mb[0, :np_], h[...],
                         wz[...], uz[...], bz[...],
                         wr[...], ur[...], br[...], wn[...], un[...], bn[...])


def _gru_out_body(ma, mb, h, wz, uz, bz, wr, ur, br, wn, un, bn, wob, oout):
    np_ = h.shape[0]
    hn = _gru_core(ma[0, :np_], mb[0, :np_], h[...],
                   wz[...], uz[...], bz[...],
                   wr[...], ur[...], br[...], wn[...], un[...], bn[...])
    oout[...] = _dot(hn, wob[...])


def _readout_body(o3, b3, bo, g, acc):
    i = pl.program_id(0)
    nb = pl.num_programs(0)
    tn = o3.shape[2]
    rows = lax.broadcasted_iota(jnp.int32, (NG, tn), 0)
    oh = (b3[0] == rows).astype(F32)
    contrib = jnp.sum(oh * o3[0], axis=1, keepdims=True)
    cnt = jnp.sum(oh, axis=1, keepdims=True)

    @pl.when(i == 0)
    def _():
        acc[...] = jnp.zeros_like(acc)

    acc[...] += contrib + cnt * bo[0, 0]

    @pl.when(i == nb - 1)
    def _():
        g[...] = acc[...]


def _make_sc_gather(n_nodes, e_pad, epw, nchunk):
    mesh = plsc.VectorSubcoreMesh(core_axis_name="c", subcore_axis_name="s",
                                  num_cores=NC, num_subcores=NS)

    @functools.partial(
        pl.kernel,
        mesh=mesh,
        out_type=jax.ShapeDtypeStruct((e_pad, H), F32),
        scratch_types=[
            pltpu.VMEM((nchunk, CH), jnp.int32),
            pltpu.VMEM((epw, H), F32),
            pltpu.VMEM_SHARED((n_nodes, H), F32),
            pltpu.SemaphoreType.DMA,
            pltpu.SemaphoreType.DMA,
        ],
        compiler_params=pltpu.CompilerParams(use_tc_tiling_on_sc=False),
    )
    def sc_gather(h_hbm, idx_hbm, out_hbm, idx_v, rows_v, h_sh, sem0, sem1):
        c = lax.axis_index("c")
        s = lax.axis_index("s")
        wid = s * NC + c

        @pl.when(s == 0)
        def _():
            pltpu.sync_copy(h_hbm, h_sh)

        pltpu.sync_copy(idx_hbm.at[wid], idx_v)
        plsc.subcore_barrier()

        def fire(j, sem):
            pltpu.async_copy(
                h_sh.at[idx_v.at[j]],
                rows_v.at[pl.ds(j * CH, CH), :],
                sem,
            )

        def drain(sem):
            pltpu.make_async_copy(
                h_sh.at[idx_v.at[0]],
                rows_v.at[pl.ds(0, CH), :],
                sem,
            ).wait()

        fire(0, sem0)
        fire(1, sem1)

        def body(jj, carry):
            j = jj * 2
            drain(sem0)
            fire(j + 2, sem0)
            drain(sem1)
            fire(j + 3, sem1)
            return carry

        lax.fori_loop(0, nchunk // 2 - 1, body, 0)
        drain(sem0)
        drain(sem1)
        pltpu.sync_copy(rows_v, out_hbm.at[pl.ds(wid * epw, epw), :])

    return sc_gather


def _make_sc_scatter(m_pad, e_pad, epw, nchunk):
    mesh = plsc.VectorSubcoreMesh(core_axis_name="c", subcore_axis_name="s",
                                  num_cores=NC, num_subcores=NS)
    rpt = m_pad // NS

    @functools.partial(
        pl.kernel,
        mesh=mesh,
        out_type=jax.ShapeDtypeStruct((NC, m_pad, H), F32),
        scratch_types=[
            pltpu.VMEM((nchunk, CH), jnp.int32),
            pltpu.VMEM((epw, H), F32),
            pltpu.VMEM_SHARED((m_pad, H), F32),
        ],
        compiler_params=pltpu.CompilerParams(use_tc_tiling_on_sc=False),
    )
    def sc_scatter(msgs_hbm, idx_hbm, zeros_hbm, out_hbm, idx_v, msg_v, macc):
        c = lax.axis_index("c")
        s = lax.axis_index("s")
        wid = s * NC + c

        @pl.when(s == 0)
        def _():
            pltpu.sync_copy(zeros_hbm, macc)

        pltpu.sync_copy(idx_hbm.at[wid], idx_v)
        pltpu.sync_copy(msgs_hbm.at[pl.ds(wid * epw, epw), :], msg_v)
        plsc.subcore_barrier()

        def body(j, carry):
            pltpu.sync_copy(
                msg_v.at[pl.ds(j * CH, CH), :],
                macc.at[idx_v.at[j]],
                add=True,
            )
            return carry

        lax.fori_loop(0, nchunk, body, 0)
        plsc.subcore_barrier()

        pltpu.sync_copy(macc.at[pl.ds(s * rpt, rpt), :],
                        msg_v.at[pl.ds(0, rpt), :])
        pltpu.sync_copy(msg_v.at[pl.ds(0, rpt), :],
                        out_hbm.at[c, pl.ds(s * rpt, rpt), :])

    return sc_scatter


def kernel(node_features, edge_features, Esrc, Etgt, batch, W_in, b_in,
           ee_W1, ee_b1, ee_W2, ee_b2, Wz, Uz, bz, Wr, Ur, br, Wn, Un, bn,
           W_out, b_out):
    n, f = node_features.shape
    e, de = edge_features.shape
    t_rounds = 3

    e_pad = ((e + NW * CH - 1) // (NW * CH)) * (NW * CH)
    epw = e_pad // NW
    nchunk = epw // CH
    m_pad = ((n + 1 + NS * PK - 1) // (NS * PK)) * (NS * PK)
    np_ = n // PK
    mp_ = m_pad // PK
    ep_ = e_pad // PK
    tn = 2000
    nb = n // tn
    tnp = np_ // nb
    tep = 512
    neb = ep_ // tep

    del tnp
    esrc_r = PK * (Esrc % np_) + Esrc // np_
    etgt_r = PK * (Etgt % np_) + Etgt // np_
    ef_p = jnp.pad(edge_features.reshape(e // PK, PK * de),
                   ((0, ep_ - e // PK), (0, 0)))
    esrc_p = jnp.pad(esrc_r, (0, e_pad - e)).reshape(NW, nchunk, CH)
    etgt_p = jnp.pad(etgt_r, (0, e_pad - e), constant_values=n).reshape(
        NW, nchunk, CH)
    zeros_m = jnp.zeros((m_pad, H), F32)
    pp = jnp.arange(n)
    batch3 = batch[(pp // PK) + np_ * (pp % PK)].reshape(nb, 1, tn)

    w1b = _kron8(ee_W1)
    w2b = _kron8(ee_W2)
    ll = jnp.arange(PK * HH)
    tt, qq = ll // (PK * H), ll % (PK * H)
    c_orig = HH * (qq // H) + H * tt + qq % H
    w2bp = w2b[:, c_orig]
    colidx = H * (qq // H) + tt
    sbp = (colidx[:, None] == jnp.arange(PK * H)[None, :]).astype(F32)
    uzb, urb, unb = _kron8(Uz), _kron8(Ur), _kron8(Un)
    wzb, wrb, wnb = _kron8(Wz), _kron8(Wr), _kron8(Wn)
    wob = _kron8(W_out)
    b_int = _tile8(b_in)
    b1t = _tile8(ee_b1)
    b2tp = _tile8(ee_b2)[:, c_orig]
    bzt, brt, bnt = _tile8(bz), _tile8(br), _tile8(bn)
    b_out2 = b_out.reshape(1, 1)

    full = lambda shape: pl.BlockSpec(shape, lambda i: tuple(0 for _ in shape))

    h = pl.pallas_call(
        _proj_body,
        grid=(1,),
        in_specs=[
            full((n, f)),
            full((f, H)),
            full((1, PK * H)),
        ],
        out_specs=full((np_, PK * H)),
        out_shape=jax.ShapeDtypeStruct((np_, PK * H), F32),
    )(node_features, W_in, b_int)

    sc_gather = _make_sc_gather(n, e_pad, epw, nchunk)
    sc_scatter = _make_sc_scatter(m_pad, e_pad, epw, nchunk)

    msgs_call = pl.pallas_call(
        _msgs_body,
        grid=(neb,),
        in_specs=[
            pl.BlockSpec((tep, PK * de), lambda i: (i, 0)),
            pl.BlockSpec((tep, PK * H), lambda i: (i, 0)),
            full((PK * de, PK * H)),
            full((1, PK * H)),
            full((PK * H, PK * HH)),
            full((1, PK * HH)),
            full((PK * HH, PK * H)),
        ],
        out_specs=pl.BlockSpec((tep, PK * H), lambda i: (i, 0)),
        out_shape=jax.ShapeDtypeStruct((ep_, PK * H), F32),
    )

    gru_in_specs = [
        pl.BlockSpec((1, mp_, PK * H), lambda i: (0, 0, 0)),
        pl.BlockSpec((1, mp_, PK * H), lambda i: (1, 0, 0)),
        full((np_, PK * H)),
        full((PK * H, PK * H)), full((PK * H, PK * H)), full((1, PK * H)),
        full((PK * H, PK * H)), full((PK * H, PK * H)), full((1, PK * H)),
        full((PK * H, PK * H)), full((PK * H, PK * H)), full((1, PK * H)),
    ]
    gru_call = pl.pallas_call(
        _gru_body,
        grid=(1,),
        in_specs=gru_in_specs,
        out_specs=full((np_, PK * H)),
        out_shape=jax.ShapeDtypeStruct((np_, PK * H), F32),
    )
    gru_out_call = pl.pallas_call(
        _gru_out_body,
        grid=(1,),
        in_specs=gru_in_specs + [full((PK * H, PK))],
        out_specs=full((np_, PK)),
        out_shape=jax.ShapeDtypeStruct((np_, PK), F32),
    )
    readout_call = pl.pallas_call(
        _readout_body,
        grid=(nb,),
        in_specs=[
            pl.BlockSpec((1, 1, tn), lambda i: (i, 0, 0)),
            pl.BlockSpec((1, 1, tn), lambda i: (i, 0, 0)),
            full((1, 1)),
        ],
        out_specs=pl.BlockSpec((NG, 1), lambda i: (0, 0)),
        out_shape=jax.ShapeDtypeStruct((NG, 1), F32),
        scratch_shapes=[pltpu.VMEM((NG, 1), F32)],
    )

    o_p = None
    for t in range(t_rounds):
        hs = sc_gather(h.reshape(n, H), esrc_p)
        msgs = msgs_call(ef_p, hs.reshape(ep_, PK * H),
                         w1b, b1t, w2bp, b2tp, sbp)
        m2 = sc_scatter(msgs.reshape(e_pad, H), etgt_p, zeros_m)
        m2p = m2.reshape(NC, mp_, PK * H)
        gru_args = (m2p, m2p, h, wzb, uzb, bzt, wrb, urb, brt, wnb, unb, bnt)
        if t < t_rounds - 1:
            h = gru_call(*gru_args)
        else:
            o_p = gru_out_call(*gru_args, wob)
    o3 = o_p.reshape(nb, 1, tn)
    return readout_call(o3, batch3, b_out2)


# speedup vs baseline: 1.3990x; 1.0431x over previous
import functools

import jax
import jax.numpy as jnp
from jax import lax
from jax.experimental import pallas as pl
from jax.experimental.pallas import tpu as pltpu
from jax.experimental.pallas import tpu_sc as plsc

F32 = jnp.float32
HIGH = lax.Precision.HIGHEST

H = 16
HH = H * H
PK = 8
NC = 2
NS = 16
NW = NC * NS
CH = 128
NG = 64


def _dot(a, b):
    return jnp.dot(a, b, preferred_element_type=F32)


def _kron8(w):
    return jnp.kron(jnp.eye(PK, dtype=w.dtype), w)


def _tile8(b):
    return jnp.tile(b.reshape(1, -1), (1, PK)).reshape(1, -1)


def _proj_body(nf, w, b, out):
    x = nf[...]
    np_ = out.shape[0]
    parts = [_dot(x[np_ * j:np_ * (j + 1), :], w[...]) for j in range(PK)]
    out[...] = jnp.concatenate(parts, axis=1) + b[...]


def _dotd(a, b):
    return jnp.dot(a, b, preferred_element_type=F32)


def _msgs_body(ef, hs, w1b, b1t, w2bp, b2tp, sbp, out):
    eh = jnp.maximum(_dotd(ef[...], w1b[...]) + b1t[...], 0.0)
    a = _dotd(eh, w2bp[...]) + b2tp[...]
    h_ = hs[...]
    hrep = jnp.concatenate([h_] * H, axis=1)
    out[...] = _dotd(a * hrep, sbp[...])


def _gru_core(ma, mb, mc, md, h, wz, uz, bz, wr, ur, br, wn, un, bn):
    m = (ma + mb) + (mc + md)
    z = jax.nn.sigmoid(_dot(m, wz) + _dot(h, uz) + bz)
    r = jax.nn.sigmoid(_dot(m, wr) + _dot(h, ur) + br)
    n = jnp.tanh(_dot(m, wn) + r * _dot(h, un) + bn)
    return (1.0 - z) * n + z * h


def _gru_body(ma, mb, mc, md, h, wz, uz, bz, wr, ur, br, wn, un, bn, out):
    np_ = h.shape[0]
    out[...] = _gru_core(ma[0, :np_], mb[0, :np_], mc[0, :np_], md[0, :np_],
                         h[...], wz[...], uz[...], bz[...],
                         wr[...], ur[...], br[...], wn[...], un[...], bn[...])


def _gru_out_body(ma, mb, mc, md, h, wz, uz, bz, wr, ur, br, wn, un, bn,
                  wob, oout):
    np_ = h.shape[0]
    hn = _gru_core(ma[0, :np_], mb[0, :np_], mc[0, :np_], md[0, :np_],
                   h[...], wz[...], uz[...], bz[...],
                   wr[...], ur[...], br[...], wn[...], un[...], bn[...])
    oout[...] = _dot(hn, wob[...])


def _readout_body(o3, b3, bo, g, acc):
    i = pl.program_id(0)
    nb = pl.num_programs(0)
    tn = o3.shape[2]
    rows = lax.broadcasted_iota(jnp.int32, (NG, tn), 0)
    oh = (b3[0] == rows).astype(F32)
    contrib = jnp.sum(oh * o3[0], axis=1, keepdims=True)
    cnt = jnp.sum(oh, axis=1, keepdims=True)

    @pl.when(i == 0)
    def _():
        acc[...] = jnp.zeros_like(acc)

    acc[...] += contrib + cnt * bo[0, 0]

    @pl.when(i == nb - 1)
    def _():
        g[...] = acc[...]


def _make_sc_gather(n_nodes, e_pad, epw, nchunk):
    mesh = plsc.VectorSubcoreMesh(core_axis_name="c", subcore_axis_name="s",
                                  num_cores=NC, num_subcores=NS)

    @functools.partial(
        pl.kernel,
        mesh=mesh,
        out_type=jax.ShapeDtypeStruct((e_pad, H), F32),
        scratch_types=[
            pltpu.VMEM((nchunk, CH), jnp.int32),
            pltpu.VMEM((epw, H), F32),
            pltpu.VMEM_SHARED((n_nodes, H), F32),
            pltpu.SemaphoreType.DMA,
            pltpu.SemaphoreType.DMA,
        ],
        compiler_params=pltpu.CompilerParams(use_tc_tiling_on_sc=False),
    )
    def sc_gather(h_hbm, idx_hbm, out_hbm, idx_v, rows_v, h_sh, sem0, sem1):
        c = lax.axis_index("c")
        s = lax.axis_index("s")
        wid = s * NC + c

        @pl.when(s == 0)
        def _():
            pltpu.sync_copy(h_hbm, h_sh)

        pltpu.sync_copy(idx_hbm.at[wid], idx_v)
        plsc.subcore_barrier()

        def fire(j, sem):
            pltpu.async_copy(
                h_sh.at[idx_v.at[j]],
                rows_v.at[pl.ds(j * CH, CH), :],
                sem,
            )

        def drain(sem):
            pltpu.make_async_copy(
                h_sh.at[idx_v.at[0]],
                rows_v.at[pl.ds(0, CH), :],
                sem,
            ).wait()

        fire(0, sem0)
        fire(1, sem1)

        def body(jj, carry):
            j = jj * 2
            drain(sem0)
            fire(j + 2, sem0)
            drain(sem1)
            fire(j + 3, sem1)
            return carry

        lax.fori_loop(0, nchunk // 2 - 1, body, 0)
        drain(sem0)
        drain(sem1)
        pltpu.sync_copy(rows_v, out_hbm.at[pl.ds(wid * epw, epw), :])

    return sc_gather


def _make_sc_scatter(m_pad, e_pad, epw, nchunk):
    mesh = plsc.VectorSubcoreMesh(core_axis_name="c", subcore_axis_name="s",
                                  num_cores=NC, num_subcores=NS)
    rpt = m_pad // NS

    @functools.partial(
        pl.kernel,
        mesh=mesh,
        out_type=jax.ShapeDtypeStruct((NC, m_pad, H), F32),
        scratch_types=[
            pltpu.VMEM((nchunk, CH), jnp.int32),
            pltpu.VMEM((epw, H), F32),
            pltpu.VMEM_SHARED((m_pad, H), F32),
        ],
        compiler_params=pltpu.CompilerParams(use_tc_tiling_on_sc=False),
    )
    def sc_scatter(msgs_hbm, idx_hbm, zeros_hbm, out_hbm, idx_v, msg_v, macc):
        c = lax.axis_index("c")
        s = lax.axis_index("s")
        wid = s * NC + c

        @pl.when(s == 0)
        def _():
            pltpu.sync_copy(zeros_hbm, macc)

        pltpu.sync_copy(idx_hbm.at[wid], idx_v)
        pltpu.sync_copy(msgs_hbm.at[pl.ds(wid * epw, epw), :], msg_v)
        plsc.subcore_barrier()

        def body(j, carry):
            pltpu.sync_copy(
                msg_v.at[pl.ds(j * CH, CH), :],
                macc.at[idx_v.at[j]],
                add=True,
            )
            return carry

        lax.fori_loop(0, nchunk, body, 0)
        plsc.subcore_barrier()

        pltpu.sync_copy(macc.at[pl.ds(s * rpt, rpt), :],
                        msg_v.at[pl.ds(0, rpt), :])
        pltpu.sync_copy(msg_v.at[pl.ds(0, rpt), :],
                        out_hbm.at[c, pl.ds(s * rpt, rpt), :])

    return sc_scatter


def kernel(node_features, edge_features, Esrc, Etgt, batch, W_in, b_in,
           ee_W1, ee_b1, ee_W2, ee_b2, Wz, Uz, bz, Wr, Ur, br, Wn, Un, bn,
           W_out, b_out):
    n, f = node_features.shape
    e, de = edge_features.shape
    t_rounds = 3

    e_pad = ((e + NW * CH - 1) // (NW * CH)) * (NW * CH)
    epw = e_pad // NW
    nchunk = epw // CH
    m_pad = ((n + 1 + NS * PK - 1) // (NS * PK)) * (NS * PK)
    np_ = n // PK
    mp_ = m_pad // PK
    ep_ = e_pad // PK
    tn = 2000
    nb = n // tn
    tnp = np_ // nb
    tep = 512
    neb = ep_ // tep

    del tnp
    esrc_r = PK * (Esrc % np_) + Esrc // np_
    etgt_r = PK * (Etgt % np_) + Etgt // np_
    ef_p = jnp.pad(edge_features.reshape(e // PK, PK * de),
                   ((0, ep_ - e // PK), (0, 0)))
    e_half = e_pad // 2
    epw_h = e_half // NW
    nchunk_h = epw_h // CH
    esrc_f = jnp.pad(esrc_r, (0, e_pad - e))
    etgt_f = jnp.pad(etgt_r, (0, e_pad - e), constant_values=n)
    esrc_pA = esrc_f[:e_half].reshape(NW, nchunk_h, CH)
    esrc_pB = esrc_f[e_half:].reshape(NW, nchunk_h, CH)
    etgt_pA = etgt_f[:e_half].reshape(NW, nchunk_h, CH)
    etgt_pB = etgt_f[e_half:].reshape(NW, nchunk_h, CH)
    zeros_m = jnp.zeros((m_pad, H), F32)
    pp = jnp.arange(n)
    batch3 = batch[(pp // PK) + np_ * (pp % PK)].reshape(nb, 1, tn)

    w1b = _kron8(ee_W1)
    w2b = _kron8(ee_W2)
    ll = jnp.arange(PK * HH)
    tt, qq = ll // (PK * H), ll % (PK * H)
    c_orig = HH * (qq // H) + H * tt + qq % H
    w2bp = w2b[:, c_orig]
    colidx = H * (qq // H) + tt
    sbp = (colidx[:, None] == jnp.arange(PK * H)[None, :]).astype(F32)
    uzb, urb, unb = _kron8(Uz), _kron8(Ur), _kron8(Un)
    wzb, wrb, wnb = _kron8(Wz), _kron8(Wr), _kron8(Wn)
    wob = _kron8(W_out)
    b_int = _tile8(b_in)
    b1t = _tile8(ee_b1)
    b2tp = _tile8(ee_b2)[:, c_orig]
    bzt, brt, bnt = _tile8(bz), _tile8(br), _tile8(bn)
    b_out2 = b_out.reshape(1, 1)

    full = lambda shape: pl.BlockSpec(shape, lambda i: tuple(0 for _ in shape))

    h = pl.pallas_call(
        _proj_body,
        grid=(1,),
        in_specs=[
            full((n, f)),
            full((f, H)),
            full((1, PK * H)),
        ],
        out_specs=full((np_, PK * H)),
        out_shape=jax.ShapeDtypeStruct((np_, PK * H), F32),
    )(node_features, W_in, b_int)

    sc_gather = _make_sc_gather(n, e_half, epw_h, nchunk_h)
    sc_scatter = _make_sc_scatter(m_pad, e_half, epw_h, nchunk_h)

    neb_h = neb // 2
    eph = e_half // PK

    def make_msgs_call(half):
        off = half * neb_h
        return pl.pallas_call(
            _msgs_body,
            grid=(neb_h,),
            in_specs=[
                pl.BlockSpec((tep, PK * de), lambda i: (i + off, 0)),
                pl.BlockSpec((tep, PK * H), lambda i: (i, 0)),
                full((PK * de, PK * H)),
                full((1, PK * H)),
                full((PK * H, PK * HH)),
                full((1, PK * HH)),
                full((PK * HH, PK * H)),
            ],
            out_specs=pl.BlockSpec((tep, PK * H), lambda i: (i, 0)),
            out_shape=jax.ShapeDtypeStruct((eph, PK * H), F32),
        )

    msgs_callA = make_msgs_call(0)
    msgs_callB = make_msgs_call(1)

    gru_in_specs = [
        pl.BlockSpec((1, mp_, PK * H), lambda i: (0, 0, 0)),
        pl.BlockSpec((1, mp_, PK * H), lambda i: (1, 0, 0)),
        pl.BlockSpec((1, mp_, PK * H), lambda i: (0, 0, 0)),
        pl.BlockSpec((1, mp_, PK * H), lambda i: (1, 0, 0)),
        full((np_, PK * H)),
        full((PK * H, PK * H)), full((PK * H, PK * H)), full((1, PK * H)),
        full((PK * H, PK * H)), full((PK * H, PK * H)), full((1, PK * H)),
        full((PK * H, PK * H)), full((PK * H, PK * H)), full((1, PK * H)),
    ]
    gru_call = pl.pallas_call(
        _gru_body,
        grid=(1,),
        in_specs=gru_in_specs,
        out_specs=full((np_, PK * H)),
        out_shape=jax.ShapeDtypeStruct((np_, PK * H), F32),
    )
    gru_out_call = pl.pallas_call(
        _gru_out_body,
        grid=(1,),
        in_specs=gru_in_specs + [full((PK * H, PK))],
        out_specs=full((np_, PK)),
        out_shape=jax.ShapeDtypeStruct((np_, PK), F32),
    )
    readout_call = pl.pallas_call(
        _readout_body,
        grid=(nb,),
        in_specs=[
            pl.BlockSpec((1, 1, tn), lambda i: (i, 0, 0)),
            pl.BlockSpec((1, 1, tn), lambda i: (i, 0, 0)),
            full((1, 1)),
        ],
        out_specs=pl.BlockSpec((NG, 1), lambda i: (0, 0)),
        out_shape=jax.ShapeDtypeStruct((NG, 1), F32),
        scratch_shapes=[pltpu.VMEM((NG, 1), F32)],
    )

    o_p = None
    for t in range(t_rounds):
        h_lin = h.reshape(n, H)
        hsA = sc_gather(h_lin, esrc_pA)
        hsB = sc_gather(h_lin, esrc_pB)
        msgsA = msgs_callA(ef_p, hsA.reshape(eph, PK * H),
                           w1b, b1t, w2bp, b2tp, sbp)
        msgsB = msgs_callB(ef_p, hsB.reshape(eph, PK * H),
                           w1b, b1t, w2bp, b2tp, sbp)
        m2A = sc_scatter(msgsA.reshape(e_half, H), etgt_pA, zeros_m)
        m2B = sc_scatter(msgsB.reshape(e_half, H), etgt_pB, zeros_m)
        mA = m2A.reshape(NC, mp_, PK * H)
        mB = m2B.reshape(NC, mp_, PK * H)
        gru_args = (mA, mA, mB, mB, h,
                    wzb, uzb, bzt, wrb, urb, brt, wnb, unb, bnt)
        if t < t_rounds - 1:
            h = gru_call(*gru_args)
        else:
            o_p = gru_out_call(*gru_args, wob)
    o3 = o_p.reshape(nb, 1, tn)
    return readout_call(o3, batch3, b_out2)
